# baseline (speedup 1.0000x reference)
import functools

import jax
import jax.numpy as jnp
from jax import lax
from jax.experimental import pallas as pl
from jax.experimental.pallas import tpu as pltpu

EMB_K = 1024
EMB_DIM = 64
BETA = 0.25
B = 16
T = 1024


def _vq_body(x_ref, w_ref, loss_ref, qst_ref, pp_ref, enc_ref,
             sq_acc, hist_acc):
    b = pl.program_id(0)

    x = x_ref[0]
    w = w_ref[...]

    xsq = jnp.sum(x * x, axis=0)
    wsq = jnp.sum(w * w, axis=1)
    xw = lax.dot_general(x.astype(jnp.bfloat16), w.astype(jnp.bfloat16),
                         (((0,), (1,)), ((), ())),
                         preferred_element_type=jnp.float32)
    dist = xsq[:, None] + wsq[None, :] - 2.0 * xw

    iota_k = lax.broadcasted_iota(jnp.int32, (T, EMB_K), 1)
    m = jnp.min(dist, axis=1, keepdims=True)
    idx = jnp.min(jnp.where(dist == m, iota_k, EMB_K), axis=1)

    onehot = (iota_k == idx[:, None]).astype(jnp.float32)
    enc_ref[...] = onehot

    q = lax.dot_general(w, onehot, (((0,), (1,)), ((), ())),
                        precision=lax.Precision.HIGHEST,
                        preferred_element_type=jnp.float32)
    qst_ref[0] = q

    step_sq = jnp.sum((q - x) ** 2)
    step_hist = jnp.sum(onehot, axis=0)[None, :]

    @pl.when(b == 0)
    def _init():
        sq_acc[0] = step_sq
        hist_acc[...] = step_hist

    @pl.when(b > 0)
    def _accum():
        sq_acc[0] += step_sq
        hist_acc[...] += step_hist

    @pl.when(b == pl.num_programs(0) - 1)
    def _finalize():
        loss_ref[0] = BETA * sq_acc[0] / float(B * T * EMB_DIM)
        avg = hist_acc[...] / float(B * T)
        pp_ref[0] = jnp.exp(-jnp.sum(avg * jnp.log(avg + 1e-10)))


@functools.partial(jax.jit, static_argnames=("interpret",))
def kernel(inputs, emb_weight, interpret=False):
    loss, qst, pp, enc = pl.pallas_call(
        _vq_body,
        grid=(B,),
        in_specs=[
            pl.BlockSpec((1, EMB_DIM, T), lambda b: (b, 0, 0)),
            pl.BlockSpec((EMB_K, EMB_DIM), lambda b: (0, 0)),
        ],
        out_specs=[
            pl.BlockSpec(memory_space=pltpu.SMEM),
            pl.BlockSpec((1, EMB_DIM, T), lambda b: (b, 0, 0)),
            pl.BlockSpec(memory_space=pltpu.SMEM),
            pl.BlockSpec((T, EMB_K), lambda b: (b, 0)),
        ],
        out_shape=[
            jax.ShapeDtypeStruct((1,), jnp.float32),
            jax.ShapeDtypeStruct((B, EMB_DIM, T), jnp.float32),
            jax.ShapeDtypeStruct((1,), jnp.float32),
            jax.ShapeDtypeStruct((B * T, EMB_K), jnp.float32),
        ],
        scratch_shapes=[
            pltpu.SMEM((1,), jnp.float32),
            pltpu.VMEM((1, EMB_K), jnp.float32),
        ],
        interpret=interpret,
    )(inputs, emb_weight)
    return (loss.reshape(()), qst, pp.reshape(()), enc)


# speedup vs baseline: 1.8183x; 1.8183x over previous
import functools

import jax
import jax.numpy as jnp
from jax import lax
from jax.experimental import pallas as pl
from jax.experimental.pallas import tpu as pltpu

EMB_K = 1024
EMB_DIM = 64
BETA = 0.25
B = 16
T = 1024


def _vq_body(x_ref, w_ref, loss_ref, qst_ref, pp_ref, enc_ref,
             sq_acc, hist_acc):
    b = pl.program_id(0)

    x = x_ref[0]
    w = w_ref[...]

    xsq = jnp.sum(x * x, axis=0)
    wsq = jnp.sum(w * w, axis=1)
    xw = lax.dot_general(x.astype(jnp.bfloat16), w.astype(jnp.bfloat16),
                         (((0,), (1,)), ((), ())),
                         preferred_element_type=jnp.float32)
    dist = xsq[:, None] + wsq[None, :] - 2.0 * xw

    iota_k = lax.broadcasted_iota(jnp.int32, (T, EMB_K), 1)
    m = jnp.min(dist, axis=1, keepdims=True)
    idx = jnp.min(jnp.where(dist == m, iota_k, EMB_K), axis=1)

    onehot = (iota_k == idx[:, None]).astype(jnp.float32)
    enc_ref[...] = onehot

    q = lax.dot_general(w.astype(jnp.bfloat16), onehot.astype(jnp.bfloat16),
                        (((0,), (1,)), ((), ())),
                        preferred_element_type=jnp.float32)
    qst_ref[0] = q

    step_sq = jnp.sum((q - x) ** 2)
    step_hist = jnp.sum(onehot, axis=0)[None, :]

    @pl.when(b == 0)
    def _init():
        sq_acc[0] = step_sq
        hist_acc[...] = step_hist

    @pl.when(b > 0)
    def _accum():
        sq_acc[0] += step_sq
        hist_acc[...] += step_hist

    @pl.when(b == pl.num_programs(0) - 1)
    def _finalize():
        loss_ref[0] = BETA * sq_acc[0] / float(B * T * EMB_DIM)
        avg = hist_acc[...] / float(B * T)
        pp_ref[0] = jnp.exp(-jnp.sum(avg * jnp.log(avg + 1e-10)))


@functools.partial(jax.jit, static_argnames=("interpret",))
def kernel(inputs, emb_weight, interpret=False):
    loss, qst, pp, enc = pl.pallas_call(
        _vq_body,
        grid=(B,),
        in_specs=[
            pl.BlockSpec((1, EMB_DIM, T), lambda b: (b, 0, 0)),
            pl.BlockSpec((EMB_K, EMB_DIM), lambda b: (0, 0)),
        ],
        out_specs=[
            pl.BlockSpec(memory_space=pltpu.SMEM),
            pl.BlockSpec((1, EMB_DIM, T), lambda b: (b, 0, 0)),
            pl.BlockSpec(memory_space=pltpu.SMEM),
            pl.BlockSpec((T, EMB_K), lambda b: (b, 0)),
        ],
        out_shape=[
            jax.ShapeDtypeStruct((1,), jnp.float32),
            jax.ShapeDtypeStruct((B, EMB_DIM, T), jnp.float32),
            jax.ShapeDtypeStruct((1,), jnp.float32),
            jax.ShapeDtypeStruct((B * T, EMB_K), jnp.float32),
        ],
        scratch_shapes=[
            pltpu.SMEM((1,), jnp.float32),
            pltpu.VMEM((1, EMB_K), jnp.float32),
        ],
        interpret=interpret,
    )(inputs, emb_weight)
    return (loss.reshape(()), qst, pp.reshape(()), enc)


# speedup vs baseline: 1.8759x; 1.0317x over previous
import functools

import jax
import jax.numpy as jnp
from jax import lax
from jax.experimental import pallas as pl
from jax.experimental.pallas import tpu as pltpu

EMB_K = 1024
EMB_DIM = 64
BETA = 0.25
B = 16
T = 1024


def _vq_body(x_ref, w_ref, loss_ref, qst_ref, pp_ref, enc_ref,
             sq_acc, hist_acc):
    b = pl.program_id(0)

    x = x_ref[0]
    w = w_ref[...]

    xsq = jnp.sum(x * x, axis=0)
    wsq = jnp.sum(w * w, axis=1)
    xwm2 = lax.dot_general(x.astype(jnp.bfloat16),
                           (-2.0 * w).astype(jnp.bfloat16),
                           (((0,), (1,)), ((), ())),
                           preferred_element_type=jnp.float32)
    dist = (xsq[:, None] + wsq[None, :]) + xwm2

    iota_k = lax.broadcasted_iota(jnp.int32, (1, EMB_K), 1).astype(jnp.float32)
    m = jnp.min(dist, axis=1, keepdims=True)
    idx = jnp.min(jnp.where(dist == m, iota_k, float(EMB_K)), axis=1)

    eqm = iota_k == idx[:, None]
    onehot = eqm.astype(jnp.float32)
    enc_ref[...] = onehot
    onehot_bf = eqm.astype(jnp.bfloat16)

    q = lax.dot_general(w.astype(jnp.bfloat16), onehot_bf,
                        (((0,), (1,)), ((), ())),
                        preferred_element_type=jnp.float32)
    qst_ref[0] = q

    step_sq = jnp.sum((q - x) ** 2)
    ones_t = jnp.ones((1, T), jnp.bfloat16)
    step_hist = lax.dot_general(ones_t, onehot_bf, (((1,), (0,)), ((), ())),
                                preferred_element_type=jnp.float32)

    @pl.when(b == 0)
    def _init():
        sq_acc[0] = step_sq
        hist_acc[...] = step_hist

    @pl.when(b > 0)
    def _accum():
        sq_acc[0] += step_sq
        hist_acc[...] += step_hist

    @pl.when(b == pl.num_programs(0) - 1)
    def _finalize():
        loss_ref[0] = BETA * sq_acc[0] / float(B * T * EMB_DIM)
        avg = hist_acc[...] / float(B * T)
        pp_ref[0] = jnp.exp(-jnp.sum(avg * jnp.log(avg + 1e-10)))


@functools.partial(jax.jit, static_argnames=("interpret",))
def kernel(inputs, emb_weight, interpret=False):
    loss, qst, pp, enc = pl.pallas_call(
        _vq_body,
        grid=(B,),
        in_specs=[
            pl.BlockSpec((1, EMB_DIM, T), lambda b: (b, 0, 0)),
            pl.BlockSpec((EMB_K, EMB_DIM), lambda b: (0, 0)),
        ],
        out_specs=[
            pl.BlockSpec(memory_space=pltpu.SMEM),
            pl.BlockSpec((1, EMB_DIM, T), lambda b: (b, 0, 0)),
            pl.BlockSpec(memory_space=pltpu.SMEM),
            pl.BlockSpec((T, EMB_K), lambda b: (b, 0)),
        ],
        out_shape=[
            jax.ShapeDtypeStruct((1,), jnp.float32),
            jax.ShapeDtypeStruct((B, EMB_DIM, T), jnp.float32),
            jax.ShapeDtypeStruct((1,), jnp.float32),
            jax.ShapeDtypeStruct((B * T, EMB_K), jnp.float32),
        ],
        scratch_shapes=[
            pltpu.SMEM((1,), jnp.float32),
            pltpu.VMEM((1, EMB_K), jnp.float32),
        ],
        interpret=interpret,
    )(inputs, emb_weight)
    return (loss.reshape(()), qst, pp.reshape(()), enc)


# speedup vs baseline: 2.0810x; 1.1093x over previous
import functools

import jax
import jax.numpy as jnp
from jax import lax
from jax.experimental import pallas as pl
from jax.experimental.pallas import tpu as pltpu

EMB_K = 1024
EMB_DIM = 64
BETA = 0.25
B = 16
T = 1024


def _vq_body(x_ref, w_ref, loss_ref, qst_ref, pp_ref, enc_ref,
             sq_acc, hist_acc, wm2_ref, wbf_ref, wsq_ref):
    b = pl.program_id(0)

    x = x_ref[0]

    @pl.when(b == 0)
    def _prep():
        w = w_ref[...]
        wm2_ref[...] = (-2.0 * w).astype(jnp.bfloat16)
        wbf_ref[...] = w.astype(jnp.bfloat16)
        wsq_ref[...] = jnp.sum(w * w, axis=1)[None, :]

    xsq = jnp.sum(x * x, axis=0)
    xwm2 = lax.dot_general(x.astype(jnp.bfloat16), wm2_ref[...],
                           (((0,), (1,)), ((), ())),
                           preferred_element_type=jnp.float32)
    dist = (xsq[:, None] + wsq_ref[...]) + xwm2

    iota_k = lax.broadcasted_iota(jnp.int32, (1, EMB_K), 1).astype(jnp.float32)
    m = jnp.min(dist, axis=1, keepdims=True)
    idx = jnp.min(jnp.where(dist == m, iota_k, float(EMB_K)), axis=1)

    eqm = iota_k == idx[:, None]
    onehot = eqm.astype(jnp.float32)
    enc_ref[...] = onehot
    onehot_bf = eqm.astype(jnp.bfloat16)

    q = lax.dot_general(wbf_ref[...], onehot_bf,
                        (((0,), (1,)), ((), ())),
                        preferred_element_type=jnp.float32)
    qst_ref[0] = q

    step_sq = jnp.sum((q - x) ** 2)
    ones_t = jnp.ones((1, T), jnp.bfloat16)
    step_hist = lax.dot_general(ones_t, onehot_bf, (((1,), (0,)), ((), ())),
                                preferred_element_type=jnp.float32)

    @pl.when(b == 0)
    def _init():
        sq_acc[0] = step_sq
        hist_acc[...] = step_hist

    @pl.when(b > 0)
    def _accum():
        sq_acc[0] += step_sq
        hist_acc[...] += step_hist

    @pl.when(b == pl.num_programs(0) - 1)
    def _finalize():
        loss_ref[0] = BETA * sq_acc[0] / float(B * T * EMB_DIM)
        avg = hist_acc[...] / float(B * T)
        pp_ref[0] = jnp.exp(-jnp.sum(avg * jnp.log(avg + 1e-10)))


@functools.partial(jax.jit, static_argnames=("interpret",))
def kernel(inputs, emb_weight, interpret=False):
    loss, qst, pp, enc = pl.pallas_call(
        _vq_body,
        grid=(B,),
        in_specs=[
            pl.BlockSpec((1, EMB_DIM, T), lambda b: (b, 0, 0)),
            pl.BlockSpec((EMB_K, EMB_DIM), lambda b: (0, 0)),
        ],
        out_specs=[
            pl.BlockSpec(memory_space=pltpu.SMEM),
            pl.BlockSpec((1, EMB_DIM, T), lambda b: (b, 0, 0)),
            pl.BlockSpec(memory_space=pltpu.SMEM),
            pl.BlockSpec((T, EMB_K), lambda b: (b, 0)),
        ],
        out_shape=[
            jax.ShapeDtypeStruct((1,), jnp.float32),
            jax.ShapeDtypeStruct((B, EMB_DIM, T), jnp.float32),
            jax.ShapeDtypeStruct((1,), jnp.float32),
            jax.ShapeDtypeStruct((B * T, EMB_K), jnp.float32),
        ],
        scratch_shapes=[
            pltpu.SMEM((1,), jnp.float32),
            pltpu.VMEM((1, EMB_K), jnp.float32),
            pltpu.VMEM((EMB_K, EMB_DIM), jnp.bfloat16),
            pltpu.VMEM((EMB_K, EMB_DIM), jnp.bfloat16),
            pltpu.VMEM((1, EMB_K), jnp.float32),
        ],
        interpret=interpret,
    )(inputs, emb_weight)
    return (loss.reshape(()), qst, pp.reshape(()), enc)
